# baseline (speedup 1.0000x reference)
import functools
import math

import jax
import jax.numpy as jnp
from jax import lax
from jax.experimental import pallas as pl
from jax.experimental.pallas import tpu as pltpu
from jax.experimental.pallas import tpu_sc as plsc

DIM = 768
DFF = 3072
NEXP = 64
NTOK = 2048
TILE = 128
NT = 80
NPAD = NT * TILE
INV_CHUNK = 512
SC_CORES = 2
SC_SUBCORES = 16
NW = SC_CORES * SC_SUBCORES

_HIGH = lax.Precision.HIGHEST
_SQRT2 = math.sqrt(2.0)


def _router_body(x_ref, wg_ref, bg_ref, pos_ref, texp_ref, ntot_ref):
    x = x_ref[...]
    logits = lax.dot_general(x, wg_ref[...], (((1,), (0,)), ((), ()))) \
        + bg_ref[...]
    m = jnp.max(logits, axis=1, keepdims=True)
    lane = lax.broadcasted_iota(jnp.int32, (NTOK, NEXP), 1)
    eid = jnp.min(jnp.where(logits == m, lane, NEXP), axis=1, keepdims=True)
    hot = (lane == eid).astype(jnp.float32)

    counts = jnp.sum(hot, axis=0, keepdims=True)
    ntiles = jnp.floor((counts + (TILE - 1)) / TILE)
    er = lax.broadcasted_iota(jnp.int32, (NEXP, NEXP), 0)
    ec = lax.broadcasted_iota(jnp.int32, (NEXP, NEXP), 1)
    strict = (er < ec).astype(jnp.float32)
    starts = lax.dot_general(ntiles, strict, (((1,), (0,)), ((), ())),
                             precision=_HIGH)
    total = jnp.sum(ntiles)
    aligned = starts * TILE

    rk_chunks = []
    rc = 256
    for c in range(NTOK // rc):
        rows = lax.broadcasted_iota(jnp.int32, (rc, NTOK), 0) + c * rc
        cols = lax.broadcasted_iota(jnp.int32, (rc, NTOK), 1)
        tril = (cols < rows).astype(jnp.float32)
        re_c = lax.dot_general(tril, hot, (((1,), (0,)), ((), ())),
                               precision=_HIGH)
        hot_c = hot[c * rc:(c + 1) * rc, :]
        rk_chunks.append(jnp.sum(re_c * hot_c, axis=1, keepdims=True))
    rank = jnp.concatenate(rk_chunks, axis=0)

    posf = jnp.sum(hot * aligned, axis=1, keepdims=True) + rank
    posi = posf.astype(jnp.int32)
    pos_ref[...] = posi

    trow = lax.broadcasted_iota(jnp.int32, (NT, NEXP), 0).astype(jnp.float32)
    teff = jnp.minimum(trow, total - 1.0)
    texp = jnp.sum((teff >= starts).astype(jnp.float32), axis=1,
                   keepdims=True) - 1.0
    texp_ref[...] = texp.astype(jnp.int32)
    ntot_ref[...] = jnp.full((1, 1), 0, jnp.int32) + total.astype(jnp.int32)


_router = pl.pallas_call(
    _router_body,
    out_shape=(
        jax.ShapeDtypeStruct((NTOK, 1), jnp.int32),
        jax.ShapeDtypeStruct((NT, 1), jnp.int32),
        jax.ShapeDtypeStruct((1, 1), jnp.int32),
    ),
)


@functools.lru_cache(maxsize=None)
def _make_sc_scatter(n_in, n_out):
    per_w = n_in // NW
    mesh = plsc.VectorSubcoreMesh(core_axis_name="c", subcore_axis_name="s",
                                  num_cores=SC_CORES, num_subcores=SC_SUBCORES)

    @functools.partial(
        pl.kernel,
        mesh=mesh,
        out_type=jax.ShapeDtypeStruct((n_out, DIM), jnp.float32),
        scratch_types=[
            pltpu.VMEM((per_w,), jnp.int32),
            pltpu.VMEM((per_w, DIM), jnp.float32),
            pltpu.SemaphoreType.DMA,
        ],
    )
    def scatter_k(table_hbm, idx_hbm, out_hbm, idx_v, rows_v, sem):
        wid = lax.axis_index("s") * SC_CORES + lax.axis_index("c")
        base = wid * per_w
        pltpu.sync_copy(idx_hbm.at[pl.ds(base, per_w)], idx_v)
        pltpu.sync_copy(table_hbm.at[pl.ds(base, per_w)], rows_v)
        pltpu.async_copy(rows_v, out_hbm.at[idx_v], sem).wait()

    return scatter_k


@functools.lru_cache(maxsize=None)
def _make_sc_gather(n_out):
    per_w = n_out // NW
    mesh = plsc.VectorSubcoreMesh(core_axis_name="c", subcore_axis_name="s",
                                  num_cores=SC_CORES, num_subcores=SC_SUBCORES)

    @functools.partial(
        pl.kernel,
        mesh=mesh,
        out_type=jax.ShapeDtypeStruct((n_out, DIM), jnp.float32),
        scratch_types=[
            pltpu.VMEM((per_w,), jnp.int32),
            pltpu.VMEM((per_w, DIM), jnp.float32),
            pltpu.SemaphoreType.DMA,
        ],
    )
    def gather_k(table_hbm, idx_hbm, out_hbm, idx_v, rows_v, sem):
        wid = lax.axis_index("s") * SC_CORES + lax.axis_index("c")
        base = wid * per_w
        pltpu.sync_copy(idx_hbm.at[pl.ds(base, per_w)], idx_v)
        pltpu.async_copy(table_hbm.at[idx_v], rows_v, sem).wait()
        pltpu.sync_copy(rows_v, out_hbm.at[pl.ds(base, per_w)])

    return gather_k


def _ffn_body(texp_ref, ntot_ref, xs_ref, w1_ref, b1_ref, w2_ref, b2_ref,
              o_ref):
    t = pl.program_id(0)

    @pl.when(t < ntot_ref[0])
    def _():
        h = lax.dot_general(xs_ref[...], w1_ref[0], (((1,), (0,)), ((), ())))
        h = h + b1_ref[0]
        h = 0.5 * h * (1.0 + lax.erf(h / _SQRT2))
        o_ref[...] = (lax.dot_general(h, w2_ref[0], (((1,), (0,)), ((), ())))
                      + b2_ref[0])


_ffn = pl.pallas_call(
    _ffn_body,
    grid_spec=pltpu.PrefetchScalarGridSpec(
        num_scalar_prefetch=2,
        grid=(NT,),
        in_specs=[
            pl.BlockSpec((TILE, DIM),
                         lambda t, texp, ntot: (jnp.minimum(t, ntot[0] - 1), 0)),
            pl.BlockSpec((1, DIM, DFF), lambda t, texp, ntot: (texp[t], 0, 0)),
            pl.BlockSpec((1, 1, DFF), lambda t, texp, ntot: (texp[t], 0, 0)),
            pl.BlockSpec((1, DFF, DIM), lambda t, texp, ntot: (texp[t], 0, 0)),
            pl.BlockSpec((1, 1, DIM), lambda t, texp, ntot: (texp[t], 0, 0)),
        ],
        out_specs=pl.BlockSpec((TILE, DIM),
                               lambda t, texp, ntot: (jnp.minimum(t, ntot[0] - 1),
                                                      0)),
    ),
    out_shape=jax.ShapeDtypeStruct((NPAD, DIM), jnp.float32),
    compiler_params=pltpu.CompilerParams(
        dimension_semantics=("arbitrary",),
    ),
)


def kernel(x, Wg, bg, W1, b1, W2, b2):
    b, t, d = x.shape
    xf = x.reshape(NTOK, DIM)
    pos, texp, ntot = _router(xf, Wg, bg.reshape(1, NEXP))
    xs = _make_sc_scatter(NTOK, NPAD)(xf, pos.reshape(NTOK))
    ys = _ffn(texp.reshape(NT), ntot.reshape(1), xs, W1,
              b1.reshape(NEXP, 1, DFF), W2, b2.reshape(NEXP, 1, DIM))
    out = _make_sc_gather(NTOK)(ys, pos.reshape(NTOK))
    return out.reshape(b, t, d)


# speedup vs baseline: 1.0562x; 1.0562x over previous
import functools
import math

import jax
import jax.numpy as jnp
from jax import lax
from jax.experimental import pallas as pl
from jax.experimental.pallas import tpu as pltpu
from jax.experimental.pallas import tpu_sc as plsc

DIM = 768
DFF = 3072
NEXP = 64
NTOK = 2048
TILE = 64
NT = 96
NPAD = NT * TILE
INV_CHUNK = 512
SC_CORES = 2
SC_SUBCORES = 16
NW = SC_CORES * SC_SUBCORES

_HIGH = lax.Precision.HIGHEST
_SQRT2 = math.sqrt(2.0)


def _router_body(x_ref, wg_ref, bg_ref, pos_ref, texp_ref, ntot_ref):
    x = x_ref[...]
    logits = lax.dot_general(x, wg_ref[...], (((1,), (0,)), ((), ()))) \
        + bg_ref[...]
    m = jnp.max(logits, axis=1, keepdims=True)
    lane = lax.broadcasted_iota(jnp.int32, (NTOK, NEXP), 1)
    eid = jnp.min(jnp.where(logits == m, lane, NEXP), axis=1, keepdims=True)
    hot = (lane == eid).astype(jnp.float32)

    counts = jnp.sum(hot, axis=0, keepdims=True)
    ntiles = jnp.floor((counts + (TILE - 1)) / TILE)
    er = lax.broadcasted_iota(jnp.int32, (NEXP, NEXP), 0)
    ec = lax.broadcasted_iota(jnp.int32, (NEXP, NEXP), 1)
    strict = (er < ec).astype(jnp.float32)
    starts = lax.dot_general(ntiles, strict, (((1,), (0,)), ((), ())),
                             precision=_HIGH)
    total = jnp.sum(ntiles)
    aligned = starts * TILE

    rk_chunks = []
    rc = 256
    for c in range(NTOK // rc):
        rows = lax.broadcasted_iota(jnp.int32, (rc, NTOK), 0) + c * rc
        cols = lax.broadcasted_iota(jnp.int32, (rc, NTOK), 1)
        tril = (cols < rows).astype(jnp.float32)
        re_c = lax.dot_general(tril, hot, (((1,), (0,)), ((), ())))
        hot_c = hot[c * rc:(c + 1) * rc, :]
        rk_chunks.append(jnp.sum(re_c * hot_c, axis=1, keepdims=True))
    rank = jnp.concatenate(rk_chunks, axis=0)

    posf = jnp.sum(hot * aligned, axis=1, keepdims=True) + rank
    posi = posf.astype(jnp.int32)
    pos_ref[...] = posi

    trow = lax.broadcasted_iota(jnp.int32, (NT, NEXP), 0).astype(jnp.float32)
    teff = jnp.minimum(trow, total - 1.0)
    texp = jnp.sum((teff >= starts).astype(jnp.float32), axis=1,
                   keepdims=True) - 1.0
    texp_ref[...] = texp.astype(jnp.int32)
    ntot_ref[...] = jnp.full((1, 1), 0, jnp.int32) + total.astype(jnp.int32)


_router = pl.pallas_call(
    _router_body,
    out_shape=(
        jax.ShapeDtypeStruct((NTOK, 1), jnp.int32),
        jax.ShapeDtypeStruct((NT, 1), jnp.int32),
        jax.ShapeDtypeStruct((1, 1), jnp.int32),
    ),
)


@functools.lru_cache(maxsize=None)
def _make_sc_scatter(n_in, n_out):
    per_w = n_in // NW
    mesh = plsc.VectorSubcoreMesh(core_axis_name="c", subcore_axis_name="s",
                                  num_cores=SC_CORES, num_subcores=SC_SUBCORES)

    @functools.partial(
        pl.kernel,
        mesh=mesh,
        out_type=jax.ShapeDtypeStruct((n_out, DIM), jnp.float32),
        scratch_types=[
            pltpu.VMEM((per_w,), jnp.int32),
            pltpu.VMEM((per_w, DIM), jnp.float32),
            pltpu.SemaphoreType.DMA,
        ],
    )
    def scatter_k(table_hbm, idx_hbm, out_hbm, idx_v, rows_v, sem):
        wid = lax.axis_index("s") * SC_CORES + lax.axis_index("c")
        base = wid * per_w
        pltpu.sync_copy(idx_hbm.at[pl.ds(base, per_w)], idx_v)
        pltpu.sync_copy(table_hbm.at[pl.ds(base, per_w)], rows_v)
        pltpu.async_copy(rows_v, out_hbm.at[idx_v], sem).wait()

    return scatter_k


@functools.lru_cache(maxsize=None)
def _make_sc_gather(n_out):
    per_w = n_out // NW
    mesh = plsc.VectorSubcoreMesh(core_axis_name="c", subcore_axis_name="s",
                                  num_cores=SC_CORES, num_subcores=SC_SUBCORES)

    @functools.partial(
        pl.kernel,
        mesh=mesh,
        out_type=jax.ShapeDtypeStruct((n_out, DIM), jnp.float32),
        scratch_types=[
            pltpu.VMEM((per_w,), jnp.int32),
            pltpu.VMEM((per_w, DIM), jnp.float32),
            pltpu.SemaphoreType.DMA,
        ],
    )
    def gather_k(table_hbm, idx_hbm, out_hbm, idx_v, rows_v, sem):
        wid = lax.axis_index("s") * SC_CORES + lax.axis_index("c")
        base = wid * per_w
        pltpu.sync_copy(idx_hbm.at[pl.ds(base, per_w)], idx_v)
        pltpu.async_copy(table_hbm.at[idx_v], rows_v, sem).wait()
        pltpu.sync_copy(rows_v, out_hbm.at[pl.ds(base, per_w)])

    return gather_k


def _ffn_body(texp_ref, ntot_ref, xs_ref, w1_ref, b1_ref, w2_ref, b2_ref,
              o_ref):
    t = pl.program_id(0)

    @pl.when(t < ntot_ref[0])
    def _():
        h = lax.dot_general(xs_ref[...], w1_ref[0], (((1,), (0,)), ((), ())))
        h = h + b1_ref[0]
        h = 0.5 * h * (1.0 + lax.erf(h / _SQRT2))
        o_ref[...] = (lax.dot_general(h, w2_ref[0], (((1,), (0,)), ((), ())))
                      + b2_ref[0])


_ffn = pl.pallas_call(
    _ffn_body,
    grid_spec=pltpu.PrefetchScalarGridSpec(
        num_scalar_prefetch=2,
        grid=(NT,),
        in_specs=[
            pl.BlockSpec((TILE, DIM),
                         lambda t, texp, ntot: (jnp.minimum(t, ntot[0] - 1), 0)),
            pl.BlockSpec((1, DIM, DFF), lambda t, texp, ntot: (texp[t], 0, 0)),
            pl.BlockSpec((1, 1, DFF), lambda t, texp, ntot: (texp[t], 0, 0)),
            pl.BlockSpec((1, DFF, DIM), lambda t, texp, ntot: (texp[t], 0, 0)),
            pl.BlockSpec((1, 1, DIM), lambda t, texp, ntot: (texp[t], 0, 0)),
        ],
        out_specs=pl.BlockSpec((TILE, DIM),
                               lambda t, texp, ntot: (jnp.minimum(t, ntot[0] - 1),
                                                      0)),
    ),
    out_shape=jax.ShapeDtypeStruct((NPAD, DIM), jnp.float32),
    compiler_params=pltpu.CompilerParams(
        dimension_semantics=("arbitrary",),
    ),
)


def kernel(x, Wg, bg, W1, b1, W2, b2):
    b, t, d = x.shape
    xf = x.reshape(NTOK, DIM)
    pos, texp, ntot = _router(xf, Wg, bg.reshape(1, NEXP))
    xs = _make_sc_scatter(NTOK, NPAD)(xf, pos.reshape(NTOK))
    ys = _ffn(texp.reshape(NT), ntot.reshape(1), xs, W1,
              b1.reshape(NEXP, 1, DFF), W2, b2.reshape(NEXP, 1, DIM))
    out = _make_sc_gather(NTOK)(ys, pos.reshape(NTOK))
    return out.reshape(b, t, d)
